# baseline (speedup 1.0000x reference)
import functools

import jax
import jax.numpy as jnp
from jax import lax
from jax.experimental import pallas as pl
from jax.experimental.pallas import tpu as pltpu
from jax.experimental.pallas import tpu_sc as plsc

N = 10000
E = 160000
SEQ = 64
FP = 8
H = 128
N_OUT = 5
HO = 128

NC, NS, L = 2, 16, 16
NW = NC * NS
CH = 128
NCHUNK = E // CH
BASE_TRIPS = NCHUNK // NW
EXTRA = NCHUNK % NW
SR = 624
TAIL = N - NS * SR
ZCH = ((0, 128), (128, 128), (256, 128), (384, 128), (512, 112))

TRIPS = 40
EPAD = NW * TRIPS * CH
NP = N + 16
TAILP = NP - NS * SR

BN = 1000
BNG = 1024
NPAD = 10240


def _sigmoid(x):
    return 0.5 * jnp.tanh(0.5 * x) + 0.5


def _leaky(x):
    return jnp.where(x >= 0, x, 0.01 * x)


def _gru_body(x_ref, wcat_ref, bias_ref, w1_ref, out_ref):
    wcat = wcat_ref[...]
    bias = bias_ref[...]
    h = jnp.zeros((x_ref.shape[0], H), jnp.float32)
    for t in range(SEQ):
        xt = x_ref[:, t * FP:(t + 1) * FP]
        cat = jnp.concatenate([h.astype(jnp.bfloat16), xt], axis=1)
        g = jnp.dot(cat, wcat, preferred_element_type=jnp.float32) + bias
        tr = jnp.tanh(g[:, :H])
        tz = jnp.tanh(g[:, H:2 * H])
        ghn = g[:, 2 * H:3 * H]
        n = jnp.tanh(g[:, 3 * H:] + ghn + tr * ghn)
        h = 0.5 * ((n + h) + tz * (h - n))
    out_ref[...] = jnp.dot(h.astype(jnp.bfloat16), w1_ref[...],
                           preferred_element_type=jnp.float32)


def _gru_xw1(x2, wcat, bias, w1T):
    return pl.pallas_call(
        _gru_body,
        grid=(NPAD // BNG,),
        in_specs=[
            pl.BlockSpec((BNG, SEQ * FP), lambda i: (i, 0)),
            pl.BlockSpec((H + FP, 4 * H), lambda i: (0, 0)),
            pl.BlockSpec((1, 4 * H), lambda i: (0, 0)),
            pl.BlockSpec((H, H), lambda i: (0, 0)),
        ],
        out_specs=pl.BlockSpec((BNG, H), lambda i: (i, 0)),
        out_shape=jax.ShapeDtypeStruct((N, H), jnp.float32),
    )(x2, wcat, bias, w1T)


CW = 16


def _counts(node_idx, edge_idx):
    mesh = plsc.VectorSubcoreMesh(
        core_axis_name="c", subcore_axis_name="s", num_cores=NC, num_subcores=NS)

    @functools.partial(
        pl.kernel,
        out_type=(jax.ShapeDtypeStruct((NC, N, CW), jnp.float32),
                  jax.ShapeDtypeStruct((NC, N, CW), jnp.float32)),
        mesh=mesh,
        scratch_types=[
            pltpu.VMEM((CH,), jnp.int32),
            pltpu.VMEM((CH,), jnp.int32),
            pltpu.VMEM((CH, CW), jnp.float32),
            pltpu.VMEM((CH, CW), jnp.float32),
            pltpu.VMEM_SHARED((N, CW), jnp.float32),
            pltpu.VMEM_SHARED((N, CW), jnp.float32),
        ],
        compiler_params=pltpu.CompilerParams(use_tc_tiling_on_sc=False),
    )
    def k(src_hbm, dst_hbm, on_hbm, oe_hbm, sidx_v, didx_v, ones_v, zero_v,
          accn_sh, acce_sh):
        cid = lax.axis_index("c")
        sid = lax.axis_index("s")
        wid = sid * NC + cid
        ones16 = jnp.full((L,), 1.0, jnp.float32)
        zeros16 = jnp.zeros((L,), jnp.float32)

        def fill(i, _):
            ones_v[i] = ones16
            zero_v[i] = zeros16
            return 0

        lax.fori_loop(0, CH, fill, 0)
        base = sid * SR
        for o, sz in ZCH:
            pltpu.sync_copy(zero_v.at[pl.ds(0, sz)], accn_sh.at[pl.ds(base + o, sz)])
            pltpu.sync_copy(zero_v.at[pl.ds(0, sz)], acce_sh.at[pl.ds(base + o, sz)])

        @pl.when(sid == NS - 1)
        def _zero_tail():
            pltpu.sync_copy(zero_v.at[pl.ds(0, TAIL)], accn_sh.at[pl.ds(NS * SR, TAIL)])
            pltpu.sync_copy(zero_v.at[pl.ds(0, TAIL)], acce_sh.at[pl.ds(NS * SR, TAIL)])

        plsc.subcore_barrier()

        trips = BASE_TRIPS + jnp.where(wid < EXTRA, 1, 0)

        def chunk(j, _):
            off = (wid + NW * j) * CH
            pltpu.sync_copy(src_hbm.at[pl.ds(off, CH)], sidx_v)
            pltpu.sync_copy(dst_hbm.at[pl.ds(off, CH)], didx_v)
            pltpu.sync_copy(ones_v, accn_sh.at[sidx_v], add=True)
            pltpu.sync_copy(ones_v, acce_sh.at[didx_v], add=True)
            return 0

        lax.fori_loop(0, trips, chunk, 0)
        plsc.subcore_barrier()
        pltpu.sync_copy(accn_sh.at[pl.ds(base, SR)], on_hbm.at[cid, pl.ds(base, SR)])
        pltpu.sync_copy(acce_sh.at[pl.ds(base, SR)], oe_hbm.at[cid, pl.ds(base, SR)])

        @pl.when(sid == NS - 1)
        def _out_tail():
            pltpu.sync_copy(accn_sh.at[pl.ds(NS * SR, TAIL)],
                            on_hbm.at[cid, pl.ds(NS * SR, TAIL)])
            pltpu.sync_copy(acce_sh.at[pl.ds(NS * SR, TAIL)],
                            oe_hbm.at[cid, pl.ds(NS * SR, TAIL)])

    return k(node_idx, edge_idx)


def _spmm(table, src_idx, dst_idx):
    mesh = plsc.VectorSubcoreMesh(
        core_axis_name="c", subcore_axis_name="s", num_cores=NC, num_subcores=NS)

    @functools.partial(
        pl.kernel,
        out_type=jax.ShapeDtypeStruct((NC, N, H), jnp.float32),
        mesh=mesh,
        scratch_types=[
            pltpu.VMEM((CH,), jnp.int32),
            pltpu.VMEM((CH,), jnp.int32),
            pltpu.VMEM((CH,), jnp.int32),
            pltpu.VMEM((CH,), jnp.int32),
            pltpu.VMEM((CH, H), jnp.float32),
            pltpu.VMEM((CH, H), jnp.float32),
            pltpu.VMEM_SHARED((N, H), jnp.float32),
            pltpu.SemaphoreType.DMA,
            pltpu.SemaphoreType.DMA,
        ],
    )
    def k(table_hbm, src_hbm, dst_hbm, out_hbm, sidx0, didx0, sidx1, didx1,
          rows0, rows1, acc_sh, sem_g, sem_s):
        cid = lax.axis_index("c")
        sid = lax.axis_index("s")
        wid = sid * NC + cid
        zeros16 = jnp.zeros((L,), jnp.float32)

        def zloop(i, _):
            r = i // (H // L)
            c = (i % (H // L)) * L
            rows0[r, pl.ds(c, L)] = zeros16
            return 0

        lax.fori_loop(0, CH * H // L, zloop, 0)
        base = sid * SR
        for o, sz in ZCH:
            pltpu.sync_copy(rows0.at[pl.ds(0, sz)], acc_sh.at[pl.ds(base + o, sz)])

        @pl.when(sid == NS - 1)
        def _zero_tail():
            pltpu.sync_copy(rows0.at[pl.ds(0, TAIL)],
                            acc_sh.at[pl.ds(NS * SR, TAIL)])

        plsc.subcore_barrier()

        def stage(g, sidx, didx, rows):
            off = (wid + NW * g) * CH
            pltpu.sync_copy(src_hbm.at[pl.ds(off, CH)], sidx)
            pltpu.sync_copy(dst_hbm.at[pl.ds(off, CH)], didx)
            pltpu.async_copy(table_hbm.at[sidx], rows, sem_g)

        stage(0, sidx0, didx0, rows0)

        def body(j, _):
            stage(2 * j + 1, sidx1, didx1, rows1)
            pltpu.make_async_copy(table_hbm.at[sidx0], rows0, sem_g).wait()
            pltpu.sync_copy(rows0, acc_sh.at[didx0], add=True)
            stage(2 * j + 2, sidx0, didx0, rows0)
            pltpu.make_async_copy(table_hbm.at[sidx1], rows1, sem_g).wait()
            pltpu.sync_copy(rows1, acc_sh.at[didx1], add=True)
            return 0

        lax.fori_loop(0, (BASE_TRIPS - 1) // 2, body, 0)
        pltpu.make_async_copy(table_hbm.at[sidx0], rows0, sem_g).wait()
        pltpu.sync_copy(rows0, acc_sh.at[didx0], add=True)

        @pl.when(wid < EXTRA)
        def _extra_chunk():
            off = (wid + NW * BASE_TRIPS) * CH
            pltpu.sync_copy(src_hbm.at[pl.ds(off, CH)], sidx1)
            pltpu.sync_copy(dst_hbm.at[pl.ds(off, CH)], didx1)
            pltpu.async_copy(table_hbm.at[sidx1], rows1, sem_g).wait()
            pltpu.sync_copy(rows1, acc_sh.at[didx1], add=True)

        plsc.subcore_barrier()
        pltpu.sync_copy(acc_sh.at[pl.ds(base, SR)], out_hbm.at[cid, pl.ds(base, SR)])

        @pl.when(sid == NS - 1)
        def _out_tail():
            pltpu.sync_copy(acc_sh.at[pl.ds(NS * SR, TAIL)],
                            out_hbm.at[cid, pl.ds(NS * SR, TAIL)])

    return k(table, src_idx, dst_idx)


def _degs_body(cn_ref, ce_ref, dn_ref, de_ref):
    dsum = cn_ref[0, :, 0:1] + cn_ref[1, :, 0:1]
    esum = ce_ref[0, :, 0:1] + ce_ref[1, :, 0:1]
    dn_ref[...] = jnp.where(dsum > 0, 1.0 / jnp.where(dsum > 0, dsum, 1.0), 0.0)
    de_ref[...] = jnp.where(esum > 0, 1.0 / jnp.where(esum > 0, esum, 1.0), 0.0)


def _degs(cn, ce):
    return pl.pallas_call(
        _degs_body,
        out_shape=(jax.ShapeDtypeStruct((N, 1), jnp.float32),
                   jax.ShapeDtypeStruct((N, 1), jnp.float32)),
    )(cn, ce)


def _scale_body(p_ref, s_ref, out_ref):
    out_ref[...] = s_ref[...] * (p_ref[0] + p_ref[1])


def _scale(p, s):
    return pl.pallas_call(
        _scale_body,
        grid=(N // BN,),
        in_specs=[
            pl.BlockSpec((NC, BN, H), lambda i: (0, i, 0)),
            pl.BlockSpec((BN, 1), lambda i: (i, 0)),
        ],
        out_specs=pl.BlockSpec((BN, H), lambda i: (i, 0)),
        out_shape=jax.ShapeDtypeStruct((N, H), jnp.float32),
    )(p, s)


def _mid_body(p_ref, s_ref, b_ref, w_ref, out_ref):
    v = s_ref[...] * (p_ref[0] + p_ref[1]) + b_ref[...]
    x1 = _leaky(v)
    out_ref[...] = jnp.dot(x1, w_ref[...], preferred_element_type=jnp.float32)


def _mid(p, s, b, wT):
    return pl.pallas_call(
        _mid_body,
        grid=(N // BN,),
        in_specs=[
            pl.BlockSpec((NC, BN, H), lambda i: (0, i, 0)),
            pl.BlockSpec((BN, 1), lambda i: (i, 0)),
            pl.BlockSpec((1, H), lambda i: (0, 0)),
            pl.BlockSpec((H, H), lambda i: (0, 0)),
        ],
        out_specs=pl.BlockSpec((BN, H), lambda i: (i, 0)),
        out_shape=jax.ShapeDtypeStruct((N, H), jnp.float32),
    )(p, s, b, wT)


def _final_body(p_ref, s_ref, b_ref, w_ref, bl_ref, out_ref):
    v = s_ref[...] * (p_ref[0] + p_ref[1]) + b_ref[...]
    x2 = _leaky(v)
    y = jnp.dot(x2, w_ref[...], preferred_element_type=jnp.float32) + bl_ref[...]
    out_ref[...] = _leaky(y)


def _final(p, s, b, wT, bl):
    return pl.pallas_call(
        _final_body,
        grid=(N // BN,),
        in_specs=[
            pl.BlockSpec((NC, BN, H), lambda i: (0, i, 0)),
            pl.BlockSpec((BN, 1), lambda i: (i, 0)),
            pl.BlockSpec((1, H), lambda i: (0, 0)),
            pl.BlockSpec((H, HO), lambda i: (0, 0)),
            pl.BlockSpec((1, HO), lambda i: (0, 0)),
        ],
        out_specs=pl.BlockSpec((BN, HO), lambda i: (i, 0)),
        out_shape=jax.ShapeDtypeStruct((N, HO), jnp.float32),
    )(p, s, b, wT, bl)


def kernel(price_input, e, concept, volumn, Wih, Whh, bih, bhh, W1, b1, W2, b2, Wl1, bl1):
    del concept, volumn
    xp = jnp.pad(price_input, ((0, 0), (0, 0), (0, FP - price_input.shape[-1])))
    x2 = xp.reshape(N, SEQ * FP).astype(jnp.bfloat16)
    wihT = jnp.pad(Wih.T, ((0, FP - Wih.shape[1]), (0, 0)))
    whhT = Whh.T
    zh = jnp.zeros((H, H), jnp.float32)
    zf = jnp.zeros((FP, H), jnp.float32)
    wcat = jnp.concatenate([
        jnp.concatenate([0.5 * whhT[:, :2 * H], 0.5 * whhT[:, 2 * H:], zh], axis=1),
        jnp.concatenate([0.5 * wihT[:, :2 * H], zf, wihT[:, 2 * H:]], axis=1),
    ], axis=0)
    bias = jnp.concatenate([0.5 * (bih[:2 * H] + bhh[:2 * H]),
                            0.5 * bhh[2 * H:], bih[2 * H:]]).reshape(1, -1)

    node_idx = e[0]
    edge_idx = e[1]

    xw1 = _gru_xw1(x2, wcat.astype(jnp.bfloat16), bias,
                   W1.T.astype(jnp.bfloat16))

    cn, ce = _counts(node_idx, edge_idx)
    dinv_c, binv_c = _degs(cn, ce)

    p1 = _spmm(xw1, node_idx, edge_idx)
    ef1 = _scale(p1, binv_c)
    p2 = _spmm(ef1, edge_idx, node_idx)
    xw2 = _mid(p2, dinv_c, b1.reshape(1, -1), W2.T)

    p3 = _spmm(xw2, node_idx, edge_idx)
    ef2 = _scale(p3, binv_c)
    p4 = _spmm(ef2, edge_idx, node_idx)

    wl1T = jnp.pad(Wl1.T, ((0, 0), (0, HO - N_OUT)))
    bl1p = jnp.pad(bl1, (0, HO - N_OUT)).reshape(1, -1)
    y = _final(p4, dinv_c, b2.reshape(1, -1), wl1T, bl1p)
    return y[:, :N_OUT]


# speedup vs baseline: 1.0524x; 1.0524x over previous
import functools

import jax
import jax.numpy as jnp
from jax import lax
from jax.experimental import pallas as pl
from jax.experimental.pallas import tpu as pltpu
from jax.experimental.pallas import tpu_sc as plsc

N = 10000
E = 160000
SEQ = 64
FP = 8
H = 128
N_OUT = 5
HO = 128

NC, NS, L = 2, 16, 16
NW = NC * NS
CH = 128
NCHUNK = E // CH
BASE_TRIPS = NCHUNK // NW
EXTRA = NCHUNK % NW
SR = 624
TAIL = N - NS * SR
ZCH = ((0, 128), (128, 128), (256, 128), (384, 128), (512, 112))

TRIPS = 40
EPAD = NW * TRIPS * CH
NP = N + 16
TAILP = NP - NS * SR

BN = 1000
BNG = 1024
NPAD = 10240


def _sigmoid(x):
    return 0.5 * jnp.tanh(0.5 * x) + 0.5


def _leaky(x):
    return jnp.where(x >= 0, x, 0.01 * x)


def _gru_body(x_ref, wcat_ref, bias_ref, w1_ref, out_ref):
    wcat = wcat_ref[...]
    bias = bias_ref[...]
    h = jnp.zeros((x_ref.shape[0], H), jnp.float32)
    for t in range(SEQ):
        xt = x_ref[:, t * FP:(t + 1) * FP]
        cat = jnp.concatenate([h.astype(jnp.bfloat16), xt], axis=1)
        g = jnp.dot(cat, wcat, preferred_element_type=jnp.float32) + bias
        tr = jnp.tanh(g[:, :H])
        tz = jnp.tanh(g[:, H:2 * H])
        ghn = g[:, 2 * H:3 * H]
        n = jnp.tanh(g[:, 3 * H:] + ghn + tr * ghn)
        h = 0.5 * ((n + h) + tz * (h - n))
    out_ref[...] = jnp.dot(h.astype(jnp.bfloat16), w1_ref[...],
                           preferred_element_type=jnp.float32)


def _gru_xw1(x2, wcat, bias, w1T):
    return pl.pallas_call(
        _gru_body,
        grid=(NPAD // BNG,),
        in_specs=[
            pl.BlockSpec((BNG, SEQ * FP), lambda i: (i, 0)),
            pl.BlockSpec((H + FP, 4 * H), lambda i: (0, 0)),
            pl.BlockSpec((1, 4 * H), lambda i: (0, 0)),
            pl.BlockSpec((H, H), lambda i: (0, 0)),
        ],
        out_specs=pl.BlockSpec((BNG, H), lambda i: (i, 0)),
        out_shape=jax.ShapeDtypeStruct((N, H), jnp.float32),
    )(x2, wcat, bias, w1T)


CW = 16


def _counts(node_idx, edge_idx):
    mesh = plsc.VectorSubcoreMesh(
        core_axis_name="c", subcore_axis_name="s", num_cores=NC, num_subcores=NS)

    @functools.partial(
        pl.kernel,
        out_type=(jax.ShapeDtypeStruct((NC, N, CW), jnp.float32),
                  jax.ShapeDtypeStruct((NC, N, CW), jnp.float32)),
        mesh=mesh,
        scratch_types=[
            pltpu.VMEM((CH,), jnp.int32),
            pltpu.VMEM((CH,), jnp.int32),
            pltpu.VMEM((CH, CW), jnp.float32),
            pltpu.VMEM((CH, CW), jnp.float32),
            pltpu.VMEM_SHARED((N, CW), jnp.float32),
            pltpu.VMEM_SHARED((N, CW), jnp.float32),
        ],
        compiler_params=pltpu.CompilerParams(use_tc_tiling_on_sc=False),
    )
    def k(src_hbm, dst_hbm, on_hbm, oe_hbm, sidx_v, didx_v, ones_v, zero_v,
          accn_sh, acce_sh):
        cid = lax.axis_index("c")
        sid = lax.axis_index("s")
        wid = sid * NC + cid
        ones16 = jnp.full((L,), 1.0, jnp.float32)
        zeros16 = jnp.zeros((L,), jnp.float32)

        def fill(i, _):
            ones_v[i] = ones16
            zero_v[i] = zeros16
            return 0

        lax.fori_loop(0, CH, fill, 0)
        base = sid * SR
        for o, sz in ZCH:
            pltpu.sync_copy(zero_v.at[pl.ds(0, sz)], accn_sh.at[pl.ds(base + o, sz)])
            pltpu.sync_copy(zero_v.at[pl.ds(0, sz)], acce_sh.at[pl.ds(base + o, sz)])

        @pl.when(sid == NS - 1)
        def _zero_tail():
            pltpu.sync_copy(zero_v.at[pl.ds(0, TAIL)], accn_sh.at[pl.ds(NS * SR, TAIL)])
            pltpu.sync_copy(zero_v.at[pl.ds(0, TAIL)], acce_sh.at[pl.ds(NS * SR, TAIL)])

        plsc.subcore_barrier()

        trips = BASE_TRIPS + jnp.where(wid < EXTRA, 1, 0)

        def chunk(j, _):
            off = (wid + NW * j) * CH
            pltpu.sync_copy(src_hbm.at[pl.ds(off, CH)], sidx_v)
            pltpu.sync_copy(dst_hbm.at[pl.ds(off, CH)], didx_v)
            pltpu.sync_copy(ones_v, accn_sh.at[sidx_v], add=True)
            pltpu.sync_copy(ones_v, acce_sh.at[didx_v], add=True)
            return 0

        lax.fori_loop(0, trips, chunk, 0)
        plsc.subcore_barrier()
        pltpu.sync_copy(accn_sh.at[pl.ds(base, SR)], on_hbm.at[cid, pl.ds(base, SR)])
        pltpu.sync_copy(acce_sh.at[pl.ds(base, SR)], oe_hbm.at[cid, pl.ds(base, SR)])

        @pl.when(sid == NS - 1)
        def _out_tail():
            pltpu.sync_copy(accn_sh.at[pl.ds(NS * SR, TAIL)],
                            on_hbm.at[cid, pl.ds(NS * SR, TAIL)])
            pltpu.sync_copy(acce_sh.at[pl.ds(NS * SR, TAIL)],
                            oe_hbm.at[cid, pl.ds(NS * SR, TAIL)])

    return k(node_idx, edge_idx)


def _spmm(table, idx_pack):
    mesh = plsc.VectorSubcoreMesh(
        core_axis_name="c", subcore_axis_name="s", num_cores=NC, num_subcores=NS)

    @functools.partial(
        pl.kernel,
        out_type=jax.ShapeDtypeStruct((NC, N, H), jnp.float32),
        mesh=mesh,
        scratch_types=[
            pltpu.VMEM((2, CH), jnp.int32),
            pltpu.VMEM((2, CH), jnp.int32),
            pltpu.VMEM((CH, H), jnp.float32),
            pltpu.VMEM((CH, H), jnp.float32),
            pltpu.VMEM_SHARED((N, H), jnp.float32),
            pltpu.SemaphoreType.DMA,
            pltpu.SemaphoreType.DMA,
        ],
    )
    def k(table_hbm, idx_hbm, out_hbm, idx0, idx1,
          rows0, rows1, acc_sh, sem_g, sem_s):
        cid = lax.axis_index("c")
        sid = lax.axis_index("s")
        wid = sid * NC + cid
        zeros16 = jnp.zeros((L,), jnp.float32)

        def zloop(i, _):
            r = i // (H // L)
            c = (i % (H // L)) * L
            rows0[r, pl.ds(c, L)] = zeros16
            return 0

        lax.fori_loop(0, CH * H // L, zloop, 0)
        base = sid * SR
        for o, sz in ZCH:
            pltpu.sync_copy(rows0.at[pl.ds(0, sz)], acc_sh.at[pl.ds(base + o, sz)])

        @pl.when(sid == NS - 1)
        def _zero_tail():
            pltpu.sync_copy(rows0.at[pl.ds(0, TAIL)],
                            acc_sh.at[pl.ds(NS * SR, TAIL)])

        plsc.subcore_barrier()

        def stage(g, idx, rows):
            pltpu.sync_copy(idx_hbm.at[pl.ds(2 * (wid + NW * g), 2)], idx)
            pltpu.async_copy(table_hbm.at[idx.at[0]], rows, sem_g)

        stage(0, idx0, rows0)

        def body(j, _):
            stage(2 * j + 1, idx1, rows1)
            pltpu.make_async_copy(table_hbm.at[idx0.at[0]], rows0, sem_g).wait()
            pltpu.sync_copy(rows0, acc_sh.at[idx0.at[1]], add=True)
            stage(2 * j + 2, idx0, rows0)
            pltpu.make_async_copy(table_hbm.at[idx1.at[0]], rows1, sem_g).wait()
            pltpu.sync_copy(rows1, acc_sh.at[idx1.at[1]], add=True)
            return 0

        lax.fori_loop(0, (BASE_TRIPS - 1) // 2, body, 0)
        pltpu.make_async_copy(table_hbm.at[idx0.at[0]], rows0, sem_g).wait()
        pltpu.sync_copy(rows0, acc_sh.at[idx0.at[1]], add=True)

        @pl.when(wid < EXTRA)
        def _extra_chunk():
            pltpu.sync_copy(idx_hbm.at[pl.ds(2 * (wid + NW * BASE_TRIPS), 2)], idx1)
            pltpu.async_copy(table_hbm.at[idx1.at[0]], rows1, sem_g).wait()
            pltpu.sync_copy(rows1, acc_sh.at[idx1.at[1]], add=True)

        plsc.subcore_barrier()
        pltpu.sync_copy(acc_sh.at[pl.ds(base, SR)], out_hbm.at[cid, pl.ds(base, SR)])

        @pl.when(sid == NS - 1)
        def _out_tail():
            pltpu.sync_copy(acc_sh.at[pl.ds(NS * SR, TAIL)],
                            out_hbm.at[cid, pl.ds(NS * SR, TAIL)])

    return k(table, idx_pack)


def _degs_body(cn_ref, ce_ref, dn_ref, de_ref):
    dsum = cn_ref[0, :, 0:1] + cn_ref[1, :, 0:1]
    esum = ce_ref[0, :, 0:1] + ce_ref[1, :, 0:1]
    dn_ref[...] = jnp.where(dsum > 0, 1.0 / jnp.where(dsum > 0, dsum, 1.0), 0.0)
    de_ref[...] = jnp.where(esum > 0, 1.0 / jnp.where(esum > 0, esum, 1.0), 0.0)


def _degs(cn, ce):
    return pl.pallas_call(
        _degs_body,
        out_shape=(jax.ShapeDtypeStruct((N, 1), jnp.float32),
                   jax.ShapeDtypeStruct((N, 1), jnp.float32)),
    )(cn, ce)


def _scale_body(p_ref, s_ref, out_ref):
    out_ref[...] = s_ref[...] * (p_ref[0] + p_ref[1])


def _scale(p, s):
    return pl.pallas_call(
        _scale_body,
        grid=(N // BN,),
        in_specs=[
            pl.BlockSpec((NC, BN, H), lambda i: (0, i, 0)),
            pl.BlockSpec((BN, 1), lambda i: (i, 0)),
        ],
        out_specs=pl.BlockSpec((BN, H), lambda i: (i, 0)),
        out_shape=jax.ShapeDtypeStruct((N, H), jnp.float32),
    )(p, s)


def _mid_body(p_ref, s_ref, b_ref, w_ref, out_ref):
    v = s_ref[...] * (p_ref[0] + p_ref[1]) + b_ref[...]
    x1 = _leaky(v)
    out_ref[...] = jnp.dot(x1, w_ref[...], preferred_element_type=jnp.float32)


def _mid(p, s, b, wT):
    return pl.pallas_call(
        _mid_body,
        grid=(N // BN,),
        in_specs=[
            pl.BlockSpec((NC, BN, H), lambda i: (0, i, 0)),
            pl.BlockSpec((BN, 1), lambda i: (i, 0)),
            pl.BlockSpec((1, H), lambda i: (0, 0)),
            pl.BlockSpec((H, H), lambda i: (0, 0)),
        ],
        out_specs=pl.BlockSpec((BN, H), lambda i: (i, 0)),
        out_shape=jax.ShapeDtypeStruct((N, H), jnp.float32),
    )(p, s, b, wT)


def _final_body(p_ref, s_ref, b_ref, w_ref, bl_ref, out_ref):
    v = s_ref[...] * (p_ref[0] + p_ref[1]) + b_ref[...]
    x2 = _leaky(v)
    y = jnp.dot(x2, w_ref[...], preferred_element_type=jnp.float32) + bl_ref[...]
    out_ref[...] = _leaky(y)


def _final(p, s, b, wT, bl):
    return pl.pallas_call(
        _final_body,
        grid=(N // BN,),
        in_specs=[
            pl.BlockSpec((NC, BN, H), lambda i: (0, i, 0)),
            pl.BlockSpec((BN, 1), lambda i: (i, 0)),
            pl.BlockSpec((1, H), lambda i: (0, 0)),
            pl.BlockSpec((H, HO), lambda i: (0, 0)),
            pl.BlockSpec((1, HO), lambda i: (0, 0)),
        ],
        out_specs=pl.BlockSpec((BN, HO), lambda i: (i, 0)),
        out_shape=jax.ShapeDtypeStruct((N, HO), jnp.float32),
    )(p, s, b, wT, bl)


def kernel(price_input, e, concept, volumn, Wih, Whh, bih, bhh, W1, b1, W2, b2, Wl1, bl1):
    del concept, volumn
    xp = jnp.pad(price_input, ((0, 0), (0, 0), (0, FP - price_input.shape[-1])))
    x2 = xp.reshape(N, SEQ * FP).astype(jnp.bfloat16)
    wihT = jnp.pad(Wih.T, ((0, FP - Wih.shape[1]), (0, 0)))
    whhT = Whh.T
    zh = jnp.zeros((H, H), jnp.float32)
    zf = jnp.zeros((FP, H), jnp.float32)
    wcat = jnp.concatenate([
        jnp.concatenate([0.5 * whhT[:, :2 * H], 0.5 * whhT[:, 2 * H:], zh], axis=1),
        jnp.concatenate([0.5 * wihT[:, :2 * H], zf, wihT[:, 2 * H:]], axis=1),
    ], axis=0)
    bias = jnp.concatenate([0.5 * (bih[:2 * H] + bhh[:2 * H]),
                            0.5 * bhh[2 * H:], bih[2 * H:]]).reshape(1, -1)

    node_idx = e[0]
    edge_idx = e[1]

    xw1 = _gru_xw1(x2, wcat.astype(jnp.bfloat16), bias,
                   W1.T.astype(jnp.bfloat16))

    cn, ce = _counts(node_idx, edge_idx)
    dinv_c, binv_c = _degs(cn, ce)

    nchunks = node_idx.reshape(NCHUNK, 1, CH)
    echunks = edge_idx.reshape(NCHUNK, 1, CH)
    packA = jnp.concatenate([nchunks, echunks], axis=1).reshape(2 * NCHUNK, CH)
    packB = jnp.concatenate([echunks, nchunks], axis=1).reshape(2 * NCHUNK, CH)

    p1 = _spmm(xw1, packA)
    ef1 = _scale(p1, binv_c)
    p2 = _spmm(ef1, packB)
    xw2 = _mid(p2, dinv_c, b1.reshape(1, -1), W2.T)

    p3 = _spmm(xw2, packA)
    ef2 = _scale(p3, binv_c)
    p4 = _spmm(ef2, packB)

    wl1T = jnp.pad(Wl1.T, ((0, 0), (0, HO - N_OUT)))
    bl1p = jnp.pad(bl1, (0, HO - N_OUT)).reshape(1, -1)
    y = _final(p4, dinv_c, b2.reshape(1, -1), wl1T, bl1p)
    return y[:, :N_OUT]
